# Initial kernel scaffold; baseline (speedup 1.0000x reference)
import functools

import jax
import jax.numpy as jnp
from jax import lax
from jax.experimental import pallas as pl
from jax.experimental.pallas import tpu as pltpu
from jax.experimental.pallas import tpu_sc as plsc

N = 10000
E = 160000
K = N // 2
R = 1000
GRID = N // R
NPAD = 10240
EPS = 1e-3


def _k1_body(x_ref, W1_ref, c1_ref, W2_ref, c2_ref, Wg1_ref, cg1_ref,
             h_ref, m1a_ref, m1b_ref):
    xb = x_ref[...]
    h1 = jnp.maximum(jnp.dot(xb, W1_ref[...], preferred_element_type=jnp.float32)
                     + c1_ref[...], 0.0)
    h = jnp.dot(h1, W2_ref[...], preferred_element_type=jnp.float32) + c2_ref[...]
    m1 = jnp.maximum(jnp.dot(h, Wg1_ref[...], preferred_element_type=jnp.float32)
                     + cg1_ref[...], 0.0)
    h_ref[...] = h
    m1a_ref[...] = m1[:, :128]
    m1b_ref[...] = m1[:, 128:]


def _k2a_body(a_ref, b_ref, h_ref, p_ref, y_ref):
    p = p_ref[...]
    pn = p * jax.lax.rsqrt(jnp.sum(p * p))
    y = (jnp.dot(a_ref[...], pn[:128], preferred_element_type=jnp.float32)
         + jnp.dot(b_ref[...], pn[128:256], preferred_element_type=jnp.float32)
         + jnp.dot(h_ref[...], pn[256:], preferred_element_type=jnp.float32))
    y_ref[...] = y


def _uge(a, b):
    off = jnp.uint32(0x80000000)
    return (lax.bitcast_convert_type(a ^ off, jnp.int32)
            >= lax.bitcast_convert_type(b ^ off, jnp.int32))


def _ugt(a, b):
    off = jnp.uint32(0x80000000)
    return (lax.bitcast_convert_type(a ^ off, jnp.int32)
            > lax.bitcast_convert_type(b ^ off, jnp.int32))


def _k2b_body(y_ref, sel_ref, q_ref):
    y = y_ref[...]
    u = lax.bitcast_convert_type(y, jnp.uint32)
    m = jnp.where(u >= jnp.uint32(0x80000000), ~u, u | jnp.uint32(0x80000000))
    row = lax.broadcasted_iota(jnp.int32, (NPAD // 128, 128), 0)
    col = lax.broadcasted_iota(jnp.int32, (NPAD // 128, 128), 1)
    idx = row * 128 + col
    valid = idx < N
    hi = jnp.where(valid, m, jnp.uint32(0))
    lo = jnp.where(valid, jnp.uint32(0xFFFFFFFF) - idx.astype(jnp.uint32),
                   jnp.uint32(0))

    def key_ge(thi, tlo):
        return _ugt(hi, thi) | (jnp.equal(hi, thi) & _uge(lo, tlo))

    def body(t, T):
        thi, tlo = T
        b = 63 - t
        in_hi = b >= 32
        amt_hi = jnp.where(in_hi, b - 32, 0).astype(jnp.uint32)
        amt_lo = jnp.where(in_hi, 0, b).astype(jnp.uint32)
        nthi = jnp.where(in_hi, thi | (jnp.uint32(1) << amt_hi), thi)
        ntlo = jnp.where(in_hi, tlo, tlo | (jnp.uint32(1) << amt_lo))
        cnt = jnp.sum(key_ge(nthi, ntlo).astype(jnp.int32))
        ok = cnt >= K
        return (jnp.where(ok, nthi, thi), jnp.where(ok, ntlo, tlo))

    thi, tlo = lax.fori_loop(0, 64, body, (jnp.uint32(0), jnp.uint32(0)))
    sel = key_ge(thi, tlo).astype(jnp.float32)
    sel_ref[...] = sel
    q_ref[...] = sel / (1.0 + jnp.exp(-y))


def _k3_body(a_ref, b_ref, h_ref, q_ref, sel_ref, c0_ref, c1_ref,
             Wg2_ref, cg2_ref, Wp1_ref, cp1_ref, Wp2_ref, cp2_ref,
             out_ref, acc_ref):
    i = pl.program_id(0)

    @pl.when(i == 0)
    def _():
        acc_ref[...] = jnp.zeros_like(acc_ref)

    q = q_ref[...][:, :1]
    xp = jnp.concatenate([a_ref[...], b_ref[...], h_ref[...]], axis=1) * q
    m2 = jnp.maximum(jnp.dot(xp, Wg2_ref[...], preferred_element_type=jnp.float32)
                     + cg2_ref[...], 0.0)
    t = sel_ref[...][:, :1] * (c0_ref[...][:, :1] + c1_ref[...][:, :1])
    ps = jnp.sum(t * m2, axis=0, keepdims=True)
    px = jnp.sum(xp, axis=0, keepdims=True)
    acc_ref[...] += jnp.concatenate([ps, px], axis=1)

    @pl.when(i == GRID - 1)
    def _():
        gsum = acc_ref[...]
        o = jnp.maximum(jnp.dot(gsum, Wp1_ref[...],
                                preferred_element_type=jnp.float32)
                        + cp1_ref[...], 0.0)
        o2 = (jnp.dot(o, Wp2_ref[...], preferred_element_type=jnp.float32)
              + cp2_ref[...])
        e = jnp.exp(o2 - jnp.max(o2, axis=1, keepdims=True))
        out_ref[...] = e / jnp.sum(e, axis=1, keepdims=True)


def _tc_k1(x, W1f, c1, W2f, c2, Wg1f, cg1):
    wspec = pl.BlockSpec((256, 256), lambda i: (0, 0))
    cspec = pl.BlockSpec((1, 256), lambda i: (0, 0))
    return pl.pallas_call(
        _k1_body,
        grid=(GRID,),
        in_specs=[pl.BlockSpec((R, 256), lambda i: (i, 0)),
                  wspec, cspec, wspec, cspec, wspec, cspec],
        out_specs=[pl.BlockSpec((R, 256), lambda i: (i, 0)),
                   pl.BlockSpec((R, 128), lambda i: (i, 0)),
                   pl.BlockSpec((R, 128), lambda i: (i, 0))],
        out_shape=[jax.ShapeDtypeStruct((N, 256), jnp.float32),
                   jax.ShapeDtypeStruct((N, 128), jnp.float32),
                   jax.ShapeDtypeStruct((N, 128), jnp.float32)],
    )(x, W1f, c1, W2f, c2, Wg1f, cg1)


def _tc_k2a(agg1a, agg1b, h, p2):
    return pl.pallas_call(
        _k2a_body,
        grid=(GRID,),
        in_specs=[pl.BlockSpec((R, 128), lambda i: (i, 0)),
                  pl.BlockSpec((R, 128), lambda i: (i, 0)),
                  pl.BlockSpec((R, 256), lambda i: (i, 0)),
                  pl.BlockSpec((512, 1), lambda i: (0, 0))],
        out_specs=pl.BlockSpec((R, 1), lambda i: (i, 0)),
        out_shape=jax.ShapeDtypeStruct((N, 1), jnp.float32),
    )(agg1a, agg1b, h, p2)


def _tc_k2b(ypad):
    return pl.pallas_call(
        _k2b_body,
        out_shape=[jax.ShapeDtypeStruct((NPAD // 128, 128), jnp.float32),
                   jax.ShapeDtypeStruct((NPAD // 128, 128), jnp.float32)],
    )(ypad)


def _tc_k3(agg1a, agg1b, h, qw, selw, cnt0, cnt1,
           Wg2f, cg2, Wp1f, cp1, Wp2f, cp2):
    return pl.pallas_call(
        _k3_body,
        grid=(GRID,),
        in_specs=[pl.BlockSpec((R, 128), lambda i: (i, 0)),
                  pl.BlockSpec((R, 128), lambda i: (i, 0)),
                  pl.BlockSpec((R, 256), lambda i: (i, 0)),
                  pl.BlockSpec((R, 16), lambda i: (i, 0)),
                  pl.BlockSpec((R, 16), lambda i: (i, 0)),
                  pl.BlockSpec((R, 16), lambda i: (i, 0)),
                  pl.BlockSpec((R, 16), lambda i: (i, 0)),
                  pl.BlockSpec((512, 256), lambda i: (0, 0)),
                  pl.BlockSpec((1, 256), lambda i: (0, 0)),
                  pl.BlockSpec((768, 256), lambda i: (0, 0)),
                  pl.BlockSpec((1, 256), lambda i: (0, 0)),
                  pl.BlockSpec((256, 10), lambda i: (0, 0)),
                  pl.BlockSpec((1, 10), lambda i: (0, 0))],
        out_specs=pl.BlockSpec((1, 10), lambda i: (0, 0)),
        out_shape=jax.ShapeDtypeStruct((1, 10), jnp.float32),
        scratch_shapes=[pltpu.VMEM((1, 768), jnp.float32)],
    )(agg1a, agg1b, h, qw, selw, cnt0, cnt1, Wg2f, cg2, Wp1f, cp1, Wp2f, cp2)


_MESH = plsc.VectorSubcoreMesh(core_axis_name="c", subcore_axis_name="s")

_EPT_A = E // 16
_RPT = N // 16


@functools.partial(
    pl.kernel, mesh=_MESH,
    out_type=[jax.ShapeDtypeStruct((N, 128), jnp.float32),
              jax.ShapeDtypeStruct((N, 128), jnp.float32)],
    scratch_types=[
        pltpu.VMEM((128,), jnp.int32),
        pltpu.VMEM((128,), jnp.int32),
        pltpu.VMEM((128, 128), jnp.float32),
        pltpu.VMEM((16,), jnp.int32),
        pltpu.VMEM((16,), jnp.int32),
        pltpu.VMEM((16, 128), jnp.float32),
        pltpu.VMEM_SHARED((N, 128), jnp.float32),
        pltpu.SemaphoreType.DMA,
    ],
)
def _sc_agg(src_hbm, dst_hbm, m1a_hbm, m1b_hbm, zero_hbm, outa, outb,
            sidx, didx, rows, sidxt, didxt, rowst, acc, sem):
    c = lax.axis_index("c")
    s = lax.axis_index("s")

    def run(tbl, out):
        pltpu.sync_copy(zero_hbm, acc.at[pl.ds(s * _RPT, _RPT)])
        plsc.subcore_barrier()
        t0 = s * _EPT_A

        def chunk(j, _):
            base = t0 + j * 128
            pltpu.sync_copy(src_hbm.at[pl.ds(base, 128)], sidx)
            pltpu.async_copy(tbl.at[sidx], rows, sem).wait()
            pltpu.sync_copy(dst_hbm.at[pl.ds(base, 128)], didx)
            pltpu.sync_copy(rows, acc.at[didx], add=True)
            return 0

        lax.fori_loop(0, _EPT_A // 128, chunk, 0)
        base = t0 + (_EPT_A // 128) * 128
        pltpu.sync_copy(src_hbm.at[pl.ds(base, 16)], sidxt)
        pltpu.async_copy(tbl.at[sidxt], rowst, sem).wait()
        pltpu.sync_copy(dst_hbm.at[pl.ds(base, 16)], didxt)
        pltpu.sync_copy(rowst, acc.at[didxt], add=True)
        plsc.subcore_barrier()
        pltpu.sync_copy(acc.at[pl.ds(s * _RPT, _RPT)],
                        out.at[pl.ds(s * _RPT, _RPT)])

    @pl.when(c == 0)
    def _():
        run(m1a_hbm, outa)

    @pl.when(c == 1)
    def _():
        run(m1b_hbm, outb)


_EPT_B = E // 32


@functools.partial(
    pl.kernel, mesh=_MESH,
    out_type=[jax.ShapeDtypeStruct((N, 16), jnp.float32),
              jax.ShapeDtypeStruct((N, 16), jnp.float32)],
    scratch_types=[
        pltpu.VMEM((128,), jnp.int32),
        pltpu.VMEM((128,), jnp.int32),
        pltpu.VMEM((128, 16), jnp.float32),
        pltpu.VMEM((8,), jnp.int32),
        pltpu.VMEM((8,), jnp.int32),
        pltpu.VMEM((8, 16), jnp.float32),
        pltpu.VMEM_SHARED((N, 16), jnp.float32),
        pltpu.SemaphoreType.DMA,
    ],
)
def _sc_cnt(src_hbm, dst_hbm, selw_hbm, zero_hbm, out0, out1,
            sidx, didx, rows, sidxt, didxt, rowst, acc, sem):
    c = lax.axis_index("c")
    s = lax.axis_index("s")
    pltpu.sync_copy(zero_hbm, acc.at[pl.ds(s * _RPT, _RPT)])
    plsc.subcore_barrier()
    e0 = c * (E // 2) + s * _EPT_B

    def chunk(j, _):
        base = e0 + j * 128
        pltpu.sync_copy(dst_hbm.at[pl.ds(base, 128)], didx)
        pltpu.async_copy(selw_hbm.at[didx], rows, sem).wait()
        pltpu.sync_copy(src_hbm.at[pl.ds(base, 128)], sidx)
        pltpu.sync_copy(rows, acc.at[sidx], add=True)
        return 0

    lax.fori_loop(0, _EPT_B // 128, chunk, 0)
    base = e0 + (_EPT_B // 128) * 128
    pltpu.sync_copy(dst_hbm.at[pl.ds(base, 8)], didxt)
    pltpu.async_copy(selw_hbm.at[didxt], rowst, sem).wait()
    pltpu.sync_copy(src_hbm.at[pl.ds(base, 8)], sidxt)
    pltpu.sync_copy(rowst, acc.at[sidxt], add=True)
    plsc.subcore_barrier()

    @pl.when(c == 0)
    def _():
        pltpu.sync_copy(acc.at[pl.ds(s * _RPT, _RPT)],
                        out0.at[pl.ds(s * _RPT, _RPT)])

    @pl.when(c == 1)
    def _():
        pltpu.sync_copy(acc.at[pl.ds(s * _RPT, _RPT)],
                        out1.at[pl.ds(s * _RPT, _RPT)])


def kernel(x, edge_index, W1, b1, g1, be1, W2, b2, g2, be2, Wg1, bg1, gg1,
           geb1, p, Wg2, bg2, gg2, geb2, Wp1, bp1, gp1, gep1, Wp2, bp2,
           gp2, gep2):
    bns = 1.0 / jnp.sqrt(jnp.float32(1.0 + EPS))
    W1f = W1 * (g1 * bns);   c1 = (b1 * g1 * bns + be1)[None, :]
    W2f = W2 * (g2 * bns);   c2 = (b2 * g2 * bns + be2)[None, :]
    Wg1f = Wg1 * (gg1 * bns); cg1 = (bg1 * gg1 * bns + geb1)[None, :]
    Wg2f = Wg2 * (gg2 * bns); cg2 = (bg2 * gg2 * bns + geb2)[None, :]
    Wp1f = Wp1 * (gp1 * bns); cp1 = (bp1 * gp1 * bns + gep1)[None, :]
    Wp2f = Wp2 * (gp2 * bns); cp2 = (bp2 * gp2 * bns + gep2)[None, :]

    src = edge_index[0]
    dst = edge_index[1]
    zeros_a = jnp.zeros((_RPT, 128), jnp.float32)
    zeros_b = jnp.zeros((_RPT, 16), jnp.float32)

    h, m1a, m1b = _tc_k1(x, W1f, c1, W2f, c2, Wg1f, cg1)
    agg1a, agg1b = _sc_agg(src, dst, m1a, m1b, zeros_a)
    y = _tc_k2a(agg1a, agg1b, h, p.reshape(512, 1))
    ypad = jnp.concatenate([y.reshape(N), jnp.zeros((NPAD - N,), jnp.float32)]
                           ).reshape(NPAD // 128, 128)
    sel, q = _tc_k2b(ypad)
    selw = jnp.broadcast_to(sel.reshape(NPAD)[:N, None], (N, 16))
    qw = jnp.broadcast_to(q.reshape(NPAD)[:N, None], (N, 16))
    cnt0, cnt1 = _sc_cnt(src, dst, selw, zeros_b)
    out = _tc_k3(agg1a, agg1b, h, qw, selw, cnt0, cnt1,
                 Wg2f, cg2, Wp1f, cp1, Wp2f, cp2)
    return out.reshape(10)


# speedup vs baseline: 13.4717x; 13.4717x over previous
import functools

import jax
import jax.numpy as jnp
from jax import lax
from jax.experimental import pallas as pl
from jax.experimental.pallas import tpu as pltpu
from jax.experimental.pallas import tpu_sc as plsc

N = 10000
E = 160000
K = N // 2
R = 1000
GRID = N // R
NPAD = 10240
EPS = 1e-3


def _k1_body(x_ref, W1_ref, c1_ref, W2_ref, c2_ref, Wg1_ref, cg1_ref,
             h_ref, m1a_ref, m1b_ref):
    xb = x_ref[...]
    h1 = jnp.maximum(jnp.dot(xb, W1_ref[...], preferred_element_type=jnp.float32)
                     + c1_ref[...], 0.0)
    h = jnp.dot(h1, W2_ref[...], preferred_element_type=jnp.float32) + c2_ref[...]
    m1 = jnp.maximum(jnp.dot(h, Wg1_ref[...], preferred_element_type=jnp.float32)
                     + cg1_ref[...], 0.0)
    h_ref[...] = h
    m1a_ref[...] = m1[:, :128]
    m1b_ref[...] = m1[:, 128:]


def _k2a_body(a_ref, b_ref, h_ref, p_ref, y_ref):
    p = p_ref[...]
    pn = p * jax.lax.rsqrt(jnp.sum(p * p))
    y = (jnp.dot(a_ref[...], pn[:128], preferred_element_type=jnp.float32)
         + jnp.dot(b_ref[...], pn[128:256], preferred_element_type=jnp.float32)
         + jnp.dot(h_ref[...], pn[256:], preferred_element_type=jnp.float32))
    y_ref[...] = y


def _uge(a, b):
    off = jnp.uint32(0x80000000)
    return (lax.bitcast_convert_type(a ^ off, jnp.int32)
            >= lax.bitcast_convert_type(b ^ off, jnp.int32))


def _ugt(a, b):
    off = jnp.uint32(0x80000000)
    return (lax.bitcast_convert_type(a ^ off, jnp.int32)
            > lax.bitcast_convert_type(b ^ off, jnp.int32))


def _k2b_body(y_ref, sel_ref, q_ref):
    y = y_ref[...]
    u = lax.bitcast_convert_type(y, jnp.uint32)
    m = jnp.where(u >= jnp.uint32(0x80000000), ~u, u | jnp.uint32(0x80000000))
    row = lax.broadcasted_iota(jnp.int32, (NPAD // 128, 128), 0)
    col = lax.broadcasted_iota(jnp.int32, (NPAD // 128, 128), 1)
    idx = row * 128 + col
    valid = idx < N
    hi = jnp.where(valid, m, jnp.uint32(0))
    lo = jnp.where(valid, jnp.uint32(0xFFFFFFFF) - idx.astype(jnp.uint32),
                   jnp.uint32(0))

    def key_ge(thi, tlo):
        return _ugt(hi, thi) | (jnp.equal(hi, thi) & _uge(lo, tlo))

    def body(t, T):
        thi, tlo = T
        b = 63 - t
        in_hi = b >= 32
        amt_hi = jnp.where(in_hi, b - 32, 0).astype(jnp.uint32)
        amt_lo = jnp.where(in_hi, 0, b).astype(jnp.uint32)
        nthi = jnp.where(in_hi, thi | (jnp.uint32(1) << amt_hi), thi)
        ntlo = jnp.where(in_hi, tlo, tlo | (jnp.uint32(1) << amt_lo))
        cnt = jnp.sum(key_ge(nthi, ntlo).astype(jnp.int32))
        ok = cnt >= K
        return (jnp.where(ok, nthi, thi), jnp.where(ok, ntlo, tlo))

    thi, tlo = lax.fori_loop(0, 64, body, (jnp.uint32(0), jnp.uint32(0)))
    sel = key_ge(thi, tlo).astype(jnp.float32)
    sel_ref[...] = sel
    q_ref[...] = sel / (1.0 + jnp.exp(-y))


def _k3_body(a_ref, b_ref, h_ref, q_ref, sel_ref, c0_ref, c1_ref,
             Wg2_ref, cg2_ref, Wp1_ref, cp1_ref, Wp2_ref, cp2_ref,
             out_ref, acc_ref):
    i = pl.program_id(0)

    @pl.when(i == 0)
    def _():
        acc_ref[...] = jnp.zeros_like(acc_ref)

    q = q_ref[...][:, :1]
    xp = jnp.concatenate([a_ref[...], b_ref[...], h_ref[...]], axis=1) * q
    m2 = jnp.maximum(jnp.dot(xp, Wg2_ref[...], preferred_element_type=jnp.float32)
                     + cg2_ref[...], 0.0)
    t = sel_ref[...][:, :1] * (c0_ref[...][:, :1] + c1_ref[...][:, :1])
    ps = jnp.sum(t * m2, axis=0, keepdims=True)
    px = jnp.sum(xp, axis=0, keepdims=True)
    acc_ref[...] += jnp.concatenate([ps, px], axis=1)

    @pl.when(i == GRID - 1)
    def _():
        gsum = acc_ref[...]
        o = jnp.maximum(jnp.dot(gsum, Wp1_ref[...],
                                preferred_element_type=jnp.float32)
                        + cp1_ref[...], 0.0)
        o2 = (jnp.dot(o, Wp2_ref[...], preferred_element_type=jnp.float32)
              + cp2_ref[...])
        e = jnp.exp(o2 - jnp.max(o2, axis=1, keepdims=True))
        out_ref[...] = e / jnp.sum(e, axis=1, keepdims=True)


def _tc_k1(x, W1f, c1, W2f, c2, Wg1f, cg1):
    wspec = pl.BlockSpec((256, 256), lambda i: (0, 0))
    cspec = pl.BlockSpec((1, 256), lambda i: (0, 0))
    return pl.pallas_call(
        _k1_body,
        grid=(GRID,),
        in_specs=[pl.BlockSpec((R, 256), lambda i: (i, 0)),
                  wspec, cspec, wspec, cspec, wspec, cspec],
        out_specs=[pl.BlockSpec((R, 256), lambda i: (i, 0)),
                   pl.BlockSpec((R, 128), lambda i: (i, 0)),
                   pl.BlockSpec((R, 128), lambda i: (i, 0))],
        out_shape=[jax.ShapeDtypeStruct((N, 256), jnp.float32),
                   jax.ShapeDtypeStruct((N, 128), jnp.float32),
                   jax.ShapeDtypeStruct((N, 128), jnp.float32)],
    )(x, W1f, c1, W2f, c2, Wg1f, cg1)


def _tc_k2a(agg1a, agg1b, h, p2):
    return pl.pallas_call(
        _k2a_body,
        grid=(GRID,),
        in_specs=[pl.BlockSpec((R, 128), lambda i: (i, 0)),
                  pl.BlockSpec((R, 128), lambda i: (i, 0)),
                  pl.BlockSpec((R, 256), lambda i: (i, 0)),
                  pl.BlockSpec((512, 1), lambda i: (0, 0))],
        out_specs=pl.BlockSpec((R, 1), lambda i: (i, 0)),
        out_shape=jax.ShapeDtypeStruct((N, 1), jnp.float32),
    )(agg1a, agg1b, h, p2)


def _tc_k2b(ypad):
    return pl.pallas_call(
        _k2b_body,
        out_shape=[jax.ShapeDtypeStruct((NPAD // 128, 128), jnp.float32),
                   jax.ShapeDtypeStruct((NPAD // 128, 128), jnp.float32)],
    )(ypad)


def _tc_k3(agg1a, agg1b, h, qw, selw, cnt0, cnt1,
           Wg2f, cg2, Wp1f, cp1, Wp2f, cp2):
    return pl.pallas_call(
        _k3_body,
        grid=(GRID,),
        in_specs=[pl.BlockSpec((R, 128), lambda i: (i, 0)),
                  pl.BlockSpec((R, 128), lambda i: (i, 0)),
                  pl.BlockSpec((R, 256), lambda i: (i, 0)),
                  pl.BlockSpec((R, 16), lambda i: (i, 0)),
                  pl.BlockSpec((R, 128), lambda i: (i, 0)),
                  pl.BlockSpec((R, 128), lambda i: (i, 0)),
                  pl.BlockSpec((R, 128), lambda i: (i, 0)),
                  pl.BlockSpec((512, 256), lambda i: (0, 0)),
                  pl.BlockSpec((1, 256), lambda i: (0, 0)),
                  pl.BlockSpec((768, 256), lambda i: (0, 0)),
                  pl.BlockSpec((1, 256), lambda i: (0, 0)),
                  pl.BlockSpec((256, 10), lambda i: (0, 0)),
                  pl.BlockSpec((1, 10), lambda i: (0, 0))],
        out_specs=pl.BlockSpec((1, 10), lambda i: (0, 0)),
        out_shape=jax.ShapeDtypeStruct((1, 10), jnp.float32),
        scratch_shapes=[pltpu.VMEM((1, 768), jnp.float32)],
    )(agg1a, agg1b, h, qw, selw, cnt0, cnt1, Wg2f, cg2, Wp1f, cp1, Wp2f, cp2)


_EPT_A = E // 16
_RPT = NPAD // 16


@functools.lru_cache(maxsize=None)
def _get_sc_agg():
  mesh = plsc.VectorSubcoreMesh(core_axis_name="c", subcore_axis_name="s")

  @functools.partial(
      pl.kernel, mesh=mesh,
      out_type=[jax.ShapeDtypeStruct((NPAD, 128), jnp.float32),
                jax.ShapeDtypeStruct((NPAD, 128), jnp.float32)],
      scratch_types=[
          pltpu.VMEM((128,), jnp.int32),
          pltpu.VMEM((128,), jnp.int32),
          pltpu.VMEM((128, 128), jnp.float32),
          pltpu.VMEM((16,), jnp.int32),
          pltpu.VMEM((16,), jnp.int32),
          pltpu.VMEM((16, 128), jnp.float32),
          pltpu.VMEM_SHARED((NPAD, 128), jnp.float32),
          pltpu.SemaphoreType.DMA,
      ],
  )
  def _sc_agg(src_hbm, dst_hbm, m1a_hbm, m1b_hbm, zero_hbm, outa, outb,
              sidx, didx, rows, sidxt, didxt, rowst, acc, sem):
    c = lax.axis_index("c")
    s = lax.axis_index("s")

    def run(tbl, out):
        pltpu.sync_copy(zero_hbm, acc.at[pl.ds(s * _RPT, _RPT)])
        plsc.subcore_barrier()
        t0 = s * _EPT_A

        def chunk(j, _):
            base = t0 + j * 128
            pltpu.sync_copy(src_hbm.at[pl.ds(base, 128)], sidx)
            pltpu.async_copy(tbl.at[sidx], rows, sem).wait()
            pltpu.sync_copy(dst_hbm.at[pl.ds(base, 128)], didx)
            pltpu.sync_copy(rows, acc.at[didx], add=True)
            return 0

        lax.fori_loop(0, _EPT_A // 128, chunk, 0)
        base = t0 + (_EPT_A // 128) * 128
        pltpu.sync_copy(src_hbm.at[pl.ds(base, 16)], sidxt)
        pltpu.async_copy(tbl.at[sidxt], rowst, sem).wait()
        pltpu.sync_copy(dst_hbm.at[pl.ds(base, 16)], didxt)
        pltpu.sync_copy(rowst, acc.at[didxt], add=True)
        plsc.subcore_barrier()
        pltpu.sync_copy(acc.at[pl.ds(s * _RPT, _RPT)],
                        out.at[pl.ds(s * _RPT, _RPT)])

    @pl.when(c == 0)
    def _():
        run(m1a_hbm, outa)

    @pl.when(c == 1)
    def _():
        run(m1b_hbm, outb)

  return _sc_agg


_EPT_B = E // 32


@functools.lru_cache(maxsize=None)
def _get_sc_cnt():
  mesh = plsc.VectorSubcoreMesh(core_axis_name="c", subcore_axis_name="s")

  @functools.partial(
      pl.kernel, mesh=mesh,
      out_type=[jax.ShapeDtypeStruct((NPAD, 128), jnp.float32),
                jax.ShapeDtypeStruct((NPAD, 128), jnp.float32)],
      scratch_types=[
          pltpu.VMEM((128,), jnp.int32),
          pltpu.VMEM((128,), jnp.int32),
          pltpu.VMEM((128, 128), jnp.float32),
          pltpu.VMEM((8,), jnp.int32),
          pltpu.VMEM((8,), jnp.int32),
          pltpu.VMEM((8, 128), jnp.float32),
          pltpu.VMEM_SHARED((NPAD, 128), jnp.float32),
          pltpu.SemaphoreType.DMA,
      ],
  )
  def _sc_cnt(src_hbm, dst_hbm, selw_hbm, zero_hbm, out0, out1,
              sidx, didx, rows, sidxt, didxt, rowst, acc, sem):
    c = lax.axis_index("c")
    s = lax.axis_index("s")
    pltpu.sync_copy(zero_hbm, acc.at[pl.ds(s * _RPT, _RPT)])
    plsc.subcore_barrier()
    e0 = c * (E // 2) + s * _EPT_B

    def chunk(j, _):
        base = e0 + j * 128
        pltpu.sync_copy(dst_hbm.at[pl.ds(base, 128)], didx)
        pltpu.async_copy(selw_hbm.at[didx], rows, sem).wait()
        pltpu.sync_copy(src_hbm.at[pl.ds(base, 128)], sidx)
        pltpu.sync_copy(rows, acc.at[sidx], add=True)
        return 0

    lax.fori_loop(0, _EPT_B // 128, chunk, 0)
    base = e0 + (_EPT_B // 128) * 128
    pltpu.sync_copy(dst_hbm.at[pl.ds(base, 8)], didxt)
    pltpu.async_copy(selw_hbm.at[didxt], rowst, sem).wait()
    pltpu.sync_copy(src_hbm.at[pl.ds(base, 8)], sidxt)
    pltpu.sync_copy(rowst, acc.at[sidxt], add=True)
    plsc.subcore_barrier()

    @pl.when(c == 0)
    def _():
        pltpu.sync_copy(acc.at[pl.ds(s * _RPT, _RPT)],
                        out0.at[pl.ds(s * _RPT, _RPT)])

    @pl.when(c == 1)
    def _():
        pltpu.sync_copy(acc.at[pl.ds(s * _RPT, _RPT)],
                        out1.at[pl.ds(s * _RPT, _RPT)])

  return _sc_cnt


def kernel(x, edge_index, W1, b1, g1, be1, W2, b2, g2, be2, Wg1, bg1, gg1,
           geb1, p, Wg2, bg2, gg2, geb2, Wp1, bp1, gp1, gep1, Wp2, bp2,
           gp2, gep2):
    bns = 1.0 / jnp.sqrt(jnp.float32(1.0 + EPS))
    W1f = W1 * (g1 * bns);   c1 = (b1 * g1 * bns + be1)[None, :]
    W2f = W2 * (g2 * bns);   c2 = (b2 * g2 * bns + be2)[None, :]
    Wg1f = Wg1 * (gg1 * bns); cg1 = (bg1 * gg1 * bns + geb1)[None, :]
    Wg2f = Wg2 * (gg2 * bns); cg2 = (bg2 * gg2 * bns + geb2)[None, :]
    Wp1f = Wp1 * (gp1 * bns); cp1 = (bp1 * gp1 * bns + gep1)[None, :]
    Wp2f = Wp2 * (gp2 * bns); cp2 = (bp2 * gp2 * bns + gep2)[None, :]

    src = edge_index[0]
    dst = edge_index[1]
    zeros_a = jnp.zeros((_RPT, 128), jnp.float32)

    h, m1a, m1b = _tc_k1(x, W1f, c1, W2f, c2, Wg1f, cg1)
    agg1a, agg1b = _get_sc_agg()(src, dst, m1a, m1b, zeros_a)
    y = _tc_k2a(agg1a, agg1b, h, p.reshape(512, 1))
    ypad = jnp.concatenate([y.reshape(N), jnp.zeros((NPAD - N,), jnp.float32)]
                           ).reshape(NPAD // 128, 128)
    sel, q = _tc_k2b(ypad)
    selw = jnp.broadcast_to(sel.reshape(NPAD)[:N, None], (N, 128))
    qw = jnp.broadcast_to(q.reshape(NPAD)[:N, None], (N, 16))
    cnt0, cnt1 = _get_sc_cnt()(src, dst, selw, zeros_a)
    out = _tc_k3(agg1a, agg1b, h, qw, selw, cnt0, cnt1,
                 Wg2f, cg2, Wp1f, cp1, Wp2f, cp2)
    return out.reshape(10)
